# Initial kernel scaffold; baseline (speedup 1.0000x reference)
import functools

import jax
import jax.numpy as jnp
from jax import lax
from jax.experimental import pallas as pl
from jax.experimental.pallas import tpu as pltpu
from jax.experimental.pallas import tpu_sc as plsc

N = 10000
E = 320000
D = 128
H = 2 * D

NC = 2
NS = 16
NW = NC * NS
EW = E // NW
CHUNK = 80
NCHUNK = EW // CHUNK
RPT = N // NS


def _sc_segment_sum(node_feat, edge_feat, src, dst, zeros):
    mesh = plsc.VectorSubcoreMesh(core_axis_name="c", subcore_axis_name="s")

    @functools.partial(
        pl.kernel,
        mesh=mesh,
        out_type=jax.ShapeDtypeStruct((NC, N, D), jnp.float32),
        scratch_types=[
            pltpu.VMEM((CHUNK,), jnp.int32),
            pltpu.VMEM((CHUNK,), jnp.int32),
            pltpu.VMEM((CHUNK, D), jnp.float32),
            pltpu.VMEM((CHUNK, D), jnp.float32),
            pltpu.SemaphoreType.DMA,
            pltpu.VMEM_SHARED((N, D), jnp.float32),
        ],
    )
    def body(node_hbm, ef_hbm, src_hbm, dst_hbm, zero_hbm, out_hbm,
             sidx_v, didx_v, nrows_v, erows_v, sem, acc):
        cid = lax.axis_index("c")
        sid = lax.axis_index("s")
        wid = cid * NS + sid

        pltpu.sync_copy(zero_hbm.at[pl.ds(sid * RPT, RPT)],
                        acc.at[pl.ds(sid * RPT, RPT)])
        plsc.subcore_barrier()

        def chunk_body(c, carry):
            base = wid * EW + c * CHUNK
            pltpu.sync_copy(src_hbm.at[pl.ds(base, CHUNK)], sidx_v)
            pltpu.sync_copy(dst_hbm.at[pl.ds(base, CHUNK)], didx_v)
            pltpu.async_copy(node_hbm.at[sidx_v], nrows_v, sem).wait()
            pltpu.sync_copy(ef_hbm.at[pl.ds(base, CHUNK)], erows_v)
            pltpu.sync_copy(nrows_v, acc.at[didx_v], add=True)
            pltpu.sync_copy(erows_v, acc.at[didx_v], add=True)
            return carry

        lax.fori_loop(0, NCHUNK, chunk_body, 0)

        plsc.subcore_barrier()
        pltpu.sync_copy(acc.at[pl.ds(sid * RPT, RPT)],
                        out_hbm.at[cid, pl.ds(sid * RPT, RPT)])

    return body(node_feat, edge_feat, src, dst, zeros)


BN = 1000


def _mlp_body(agg_ref, w1_ref, b1_ref, w2_ref, b2_ref, out_ref):
    a = agg_ref[0] + agg_ref[1]
    h = jnp.maximum(
        jnp.dot(a, w1_ref[...], preferred_element_type=jnp.float32)
        + b1_ref[...], 0.0)
    out_ref[...] = (
        jnp.dot(h, w2_ref[...], preferred_element_type=jnp.float32)
        + b2_ref[...])


def _mlp(partials, W1, b1, W2, b2):
    return pl.pallas_call(
        _mlp_body,
        grid=(N // BN,),
        in_specs=[
            pl.BlockSpec((NC, BN, D), lambda i: (0, i, 0)),
            pl.BlockSpec((D, H), lambda i: (0, 0)),
            pl.BlockSpec((1, H), lambda i: (0, 0)),
            pl.BlockSpec((H, D), lambda i: (0, 0)),
            pl.BlockSpec((1, D), lambda i: (0, 0)),
        ],
        out_specs=pl.BlockSpec((BN, D), lambda i: (i, 0)),
        out_shape=jax.ShapeDtypeStruct((N, D), jnp.float32),
    )(partials, W1, b1, W2, b2)


@jax.jit
def kernel(node_feat, edge_feat, edge_index, W1, b1, W2, b2):
    src = edge_index[0]
    dst = edge_index[1]
    zeros = jnp.zeros((N, D), jnp.float32)
    partials = _sc_segment_sum(node_feat, edge_feat, src, dst, zeros)
    return _mlp(partials, W1, b1.reshape(1, H), W2, b2.reshape(1, D))


# speedup vs baseline: 3.5983x; 3.5983x over previous
import functools

import jax
import jax.numpy as jnp
from jax import lax
from jax.experimental import pallas as pl
from jax.experimental.pallas import tpu as pltpu
from jax.experimental.pallas import tpu_sc as plsc

N = 10000
E = 320000
D = 128
H = 2 * D

NC = 2
NS = 16
NW = NC * NS
EW = E // NW
CHUNK = 80
NCHUNK = EW // CHUNK
RPT = 624
REM = N - NS * RPT


def _sc_segment_sum(node_feat, edge_feat, src, dst, zeros):
    mesh = plsc.VectorSubcoreMesh(core_axis_name="c", subcore_axis_name="s")

    @functools.partial(
        pl.kernel,
        mesh=mesh,
        out_type=jax.ShapeDtypeStruct((NC, N, D), jnp.float32),
        scratch_types=[
            pltpu.VMEM((CHUNK,), jnp.int32),
            pltpu.VMEM((CHUNK,), jnp.int32),
            pltpu.VMEM((CHUNK, D), jnp.float32),
            pltpu.VMEM((CHUNK, D), jnp.float32),
            pltpu.SemaphoreType.DMA,
            pltpu.VMEM_SHARED((N, D), jnp.float32),
        ],
    )
    def body(node_hbm, ef_hbm, src_hbm, dst_hbm, zero_hbm, out_hbm,
             sidx_v, didx_v, nrows_v, erows_v, sem, acc):
        cid = lax.axis_index("c")
        sid = lax.axis_index("s")
        wid = cid * NS + sid

        pltpu.sync_copy(zero_hbm.at[pl.ds(sid * RPT, RPT)],
                        acc.at[pl.ds(sid * RPT, RPT)])

        @pl.when(sid == NS - 1)
        def _():
            pltpu.sync_copy(zero_hbm.at[pl.ds(NS * RPT, REM)],
                            acc.at[pl.ds(NS * RPT, REM)])

        plsc.subcore_barrier()

        def chunk_body(c, carry):
            base = wid * EW + c * CHUNK
            pltpu.sync_copy(src_hbm.at[pl.ds(base, CHUNK)], sidx_v)
            pltpu.sync_copy(dst_hbm.at[pl.ds(base, CHUNK)], didx_v)
            pltpu.async_copy(node_hbm.at[sidx_v], nrows_v, sem).wait()
            pltpu.sync_copy(ef_hbm.at[pl.ds(base, CHUNK)], erows_v)
            pltpu.sync_copy(nrows_v, acc.at[didx_v], add=True)
            pltpu.sync_copy(erows_v, acc.at[didx_v], add=True)
            return carry

        lax.fori_loop(0, NCHUNK, chunk_body, 0)

        plsc.subcore_barrier()
        pltpu.sync_copy(acc.at[pl.ds(sid * RPT, RPT)],
                        out_hbm.at[cid, pl.ds(sid * RPT, RPT)])

        @pl.when(sid == NS - 1)
        def _():
            pltpu.sync_copy(acc.at[pl.ds(NS * RPT, REM)],
                            out_hbm.at[cid, pl.ds(NS * RPT, REM)])

    return body(node_feat, edge_feat, src, dst, zeros)


BN = 1000


def _mlp_body(agg_ref, w1_ref, b1_ref, w2_ref, b2_ref, out_ref):
    a = agg_ref[0] + agg_ref[1]
    h = jnp.maximum(
        jnp.dot(a, w1_ref[...], preferred_element_type=jnp.float32)
        + b1_ref[...], 0.0)
    out_ref[...] = (
        jnp.dot(h, w2_ref[...], preferred_element_type=jnp.float32)
        + b2_ref[...])


def _mlp(partials, W1, b1, W2, b2):
    return pl.pallas_call(
        _mlp_body,
        grid=(N // BN,),
        in_specs=[
            pl.BlockSpec((NC, BN, D), lambda i: (0, i, 0)),
            pl.BlockSpec((D, H), lambda i: (0, 0)),
            pl.BlockSpec((1, H), lambda i: (0, 0)),
            pl.BlockSpec((H, D), lambda i: (0, 0)),
            pl.BlockSpec((1, D), lambda i: (0, 0)),
        ],
        out_specs=pl.BlockSpec((BN, D), lambda i: (i, 0)),
        out_shape=jax.ShapeDtypeStruct((N, D), jnp.float32),
    )(partials, W1, b1, W2, b2)


@jax.jit
def kernel(node_feat, edge_feat, edge_index, W1, b1, W2, b2):
    src = edge_index[0]
    dst = edge_index[1]
    zeros = jnp.zeros((N, D), jnp.float32)
    partials = _sc_segment_sum(node_feat, edge_feat, src, dst, zeros)
    return _mlp(partials, W1, b1.reshape(1, H), W2, b2.reshape(1, D))


# speedup vs baseline: 8.3118x; 2.3099x over previous
import functools

import jax
import jax.numpy as jnp
from jax import lax
from jax.experimental import pallas as pl
from jax.experimental.pallas import tpu as pltpu
from jax.experimental.pallas import tpu_sc as plsc

N = 10000
E = 320000
D = 128
H = 2 * D

NC = 2
NS = 16
NW = NC * NS
EW = E // NW
CHUNK = 80
NCHUNK = EW // CHUNK
RPT = 624
REM = N - NS * RPT


NBUF = 2
IBUF = 2 * NBUF


def _sc_segment_sum(node_feat, edge_feat, src2, dst2, zeros):
    mesh = plsc.VectorSubcoreMesh(core_axis_name="c", subcore_axis_name="s")

    @functools.partial(
        pl.kernel,
        mesh=mesh,
        out_type=jax.ShapeDtypeStruct((NC, N, D), jnp.float32),
        scratch_types=[
            [pltpu.VMEM((CHUNK,), jnp.int32)] * IBUF,
            [pltpu.VMEM((CHUNK,), jnp.int32)] * IBUF,
            [pltpu.VMEM((CHUNK, D), jnp.float32)] * NBUF,
            [pltpu.VMEM((CHUNK, D), jnp.float32)] * NBUF,
            [pltpu.SemaphoreType.DMA] * IBUF,
            [pltpu.SemaphoreType.DMA] * NBUF,
            pltpu.VMEM_SHARED((N, D), jnp.float32),
        ],
    )
    def body(node_hbm, ef_hbm, src_hbm, dst_hbm, zero_hbm, out_hbm,
             sidx_v, didx_v, nrows_v, erows_v, isems, dsems, acc):
        cid = lax.axis_index("c")
        sid = lax.axis_index("s")
        wid = cid * NS + sid

        pltpu.sync_copy(zero_hbm.at[pl.ds(sid * RPT, RPT)],
                        acc.at[pl.ds(sid * RPT, RPT)])

        @pl.when(sid == NS - 1)
        def _():
            pltpu.sync_copy(zero_hbm.at[pl.ds(NS * RPT, REM)],
                            acc.at[pl.ds(NS * RPT, REM)])

        plsc.subcore_barrier()

        def issue_idx(c, j):
            base = wid * EW + c * CHUNK
            pltpu.async_copy(src_hbm.at[pl.ds(base, CHUNK)], sidx_v[j],
                             isems[j])
            pltpu.async_copy(dst_hbm.at[pl.ds(base, CHUNK)], didx_v[j],
                             isems[j])

        def wait_idx(j):
            pltpu.make_async_copy(src_hbm.at[pl.ds(0, CHUNK)],
                                  sidx_v[j], isems[j]).wait()
            pltpu.make_async_copy(src_hbm.at[pl.ds(0, CHUNK)],
                                  didx_v[j], isems[j]).wait()

        def issue_data(c, b, j):
            base = wid * EW + c * CHUNK
            pltpu.async_copy(node_hbm.at[sidx_v[j]], nrows_v[b],
                             dsems[b])
            pltpu.async_copy(ef_hbm.at[pl.ds(base, CHUNK)], erows_v[b],
                             dsems[b])

        def wait_data(b):
            pltpu.make_async_copy(ef_hbm.at[pl.ds(0, CHUNK)],
                                  nrows_v[b], dsems[b]).wait()
            pltpu.make_async_copy(ef_hbm.at[pl.ds(0, CHUNK)],
                                  erows_v[b], dsems[b]).wait()

        for k in range(IBUF):
            issue_idx(k, k)
        for k in range(NBUF):
            wait_idx(k)
            issue_data(k, k, k)

        def group_body(g, carry):
            for k in range(IBUF):
                c = g * IBUF + k
                b = k % NBUF
                j = k

                @pl.when(c < NCHUNK)
                def _():
                    wait_data(b)
                    pltpu.sync_copy(nrows_v[b], acc.at[didx_v[j]],
                                    add=True)
                    pltpu.sync_copy(erows_v[b], acc.at[didx_v[j]],
                                    add=True)

                    @pl.when(c + IBUF < NCHUNK)
                    def _():
                        issue_idx(c + IBUF, j)

                    @pl.when(c + NBUF < NCHUNK)
                    def _():
                        jn = (k + NBUF) % IBUF
                        wait_idx(jn)
                        issue_data(c + NBUF, b, jn)

            return carry

        lax.fori_loop(0, (NCHUNK + IBUF - 1) // IBUF, group_body, 0)

        plsc.subcore_barrier()
        pltpu.sync_copy(acc.at[pl.ds(sid * RPT, RPT)],
                        out_hbm.at[cid, pl.ds(sid * RPT, RPT)])

        @pl.when(sid == NS - 1)
        def _():
            pltpu.sync_copy(acc.at[pl.ds(NS * RPT, REM)],
                            out_hbm.at[cid, pl.ds(NS * RPT, REM)])

    return body(node_feat, edge_feat, src2, dst2, zeros)


BN = 1000


def _mlp_body(agg_ref, w1_ref, b1_ref, w2_ref, b2_ref, out_ref):
    a = agg_ref[0] + agg_ref[1]
    h = jnp.maximum(
        jnp.dot(a, w1_ref[...], preferred_element_type=jnp.float32)
        + b1_ref[...], 0.0)
    out_ref[...] = (
        jnp.dot(h, w2_ref[...], preferred_element_type=jnp.float32)
        + b2_ref[...])


def _mlp(partials, W1, b1, W2, b2):
    return pl.pallas_call(
        _mlp_body,
        grid=(N // BN,),
        in_specs=[
            pl.BlockSpec((NC, BN, D), lambda i: (0, i, 0)),
            pl.BlockSpec((D, H), lambda i: (0, 0)),
            pl.BlockSpec((1, H), lambda i: (0, 0)),
            pl.BlockSpec((H, D), lambda i: (0, 0)),
            pl.BlockSpec((1, D), lambda i: (0, 0)),
        ],
        out_specs=pl.BlockSpec((BN, D), lambda i: (i, 0)),
        out_shape=jax.ShapeDtypeStruct((N, D), jnp.float32),
    )(partials, W1, b1, W2, b2)


@jax.jit
def kernel(node_feat, edge_feat, edge_index, W1, b1, W2, b2):
    src2 = edge_index[0]
    dst2 = edge_index[1]
    zeros = jnp.zeros((N, D), jnp.float32)
    partials = _sc_segment_sum(node_feat, edge_feat, src2, dst2, zeros)
    return _mlp(partials, W1, b1.reshape(1, H), W2, b2.reshape(1, D))


# speedup vs baseline: 8.9517x; 1.0770x over previous
import functools

import jax
import jax.numpy as jnp
from jax import lax
from jax.experimental import pallas as pl
from jax.experimental.pallas import tpu as pltpu
from jax.experimental.pallas import tpu_sc as plsc

N = 10000
E = 320000
D = 128
H = 2 * D

NC = 2
NS = 16
NW = NC * NS
EW = E // NW
CHUNK = 80
NCHUNK = EW // CHUNK
RPT = 624
REM = N - NS * RPT


NBUF = 4
IBUF = 8


def _sc_segment_sum(node_feat, edge_feat, src2, dst2, zeros):
    mesh = plsc.VectorSubcoreMesh(core_axis_name="c", subcore_axis_name="s")

    @functools.partial(
        pl.kernel,
        mesh=mesh,
        out_type=jax.ShapeDtypeStruct((NC, N, D), jnp.float32),
        scratch_types=[
            [pltpu.VMEM((CHUNK,), jnp.int32)] * IBUF,
            [pltpu.VMEM((CHUNK,), jnp.int32)] * IBUF,
            [pltpu.VMEM((CHUNK, D), jnp.float32)] * NBUF,
            [pltpu.SemaphoreType.DMA] * IBUF,
            [pltpu.SemaphoreType.DMA] * NBUF,
            [pltpu.SemaphoreType.DMA] * NBUF,
            [pltpu.SemaphoreType.DMA] * NBUF,
            pltpu.VMEM_SHARED((N, D), jnp.float32),
        ],
    )
    def body(node_hbm, ef_hbm, src_hbm, dst_hbm, zero_hbm, out_hbm,
             sidx_v, didx_v, mrows_v, isems, lsems, gsems, ssems, acc):
        cid = lax.axis_index("c")
        sid = lax.axis_index("s")
        wid = cid * NS + sid

        pltpu.sync_copy(zero_hbm.at[pl.ds(sid * RPT, RPT)],
                        acc.at[pl.ds(sid * RPT, RPT)])

        @pl.when(sid == NS - 1)
        def _():
            pltpu.sync_copy(zero_hbm.at[pl.ds(NS * RPT, REM)],
                            acc.at[pl.ds(NS * RPT, REM)])

        plsc.subcore_barrier()

        def issue_idx(c, j):
            base = wid * EW + c * CHUNK
            pltpu.async_copy(src_hbm.at[pl.ds(base, CHUNK)], sidx_v[j],
                             isems[j])
            pltpu.async_copy(dst_hbm.at[pl.ds(base, CHUNK)], didx_v[j],
                             isems[j])

        def wait_idx(j):
            pltpu.make_async_copy(src_hbm.at[pl.ds(0, CHUNK)],
                                  sidx_v[j], isems[j]).wait()
            pltpu.make_async_copy(src_hbm.at[pl.ds(0, CHUNK)],
                                  didx_v[j], isems[j]).wait()

        def issue_eload(c, b):
            base = wid * EW + c * CHUNK
            pltpu.async_copy(ef_hbm.at[pl.ds(base, CHUNK)], mrows_v[b],
                             lsems[b])

        def wait_eload(b):
            pltpu.make_async_copy(ef_hbm.at[pl.ds(0, CHUNK)],
                                  mrows_v[b], lsems[b]).wait()

        def issue_gadd(b, j):
            pltpu.async_copy(node_hbm.at[sidx_v[j]], mrows_v[b], gsems[b],
                             add=True)

        def wait_gadd(b):
            pltpu.make_async_copy(ef_hbm.at[pl.ds(0, CHUNK)],
                                  mrows_v[b], gsems[b]).wait()

        def issue_scatter(b, j):
            pltpu.async_copy(mrows_v[b], acc.at[didx_v[j]], ssems[b],
                             add=True)

        def wait_scatter(b):
            pltpu.make_async_copy(ef_hbm.at[pl.ds(0, CHUNK)],
                                  mrows_v[b], ssems[b]).wait()

        for k in range(IBUF):
            issue_idx(k, k)
        wait_idx(0)
        issue_eload(0, 0)
        wait_eload(0)
        issue_gadd(0, 0)
        issue_eload(1, 1)

        def group_body(g, carry):
            for k in range(IBUF):
                c = g * IBUF + k
                b = k % NBUF

                @pl.when(c < NCHUNK)
                def _():
                    b2 = (k + 2) % NBUF
                    b1 = (k + 1) % NBUF
                    j1 = (k + 1) % IBUF

                    @pl.when(c + 2 < NCHUNK)
                    def _():
                        @pl.when(c >= 2)
                        def _():
                            wait_scatter(b2)
                        issue_eload(c + 2, b2)

                    @pl.when(c + 1 < NCHUNK)
                    def _():
                        wait_eload(b1)
                        wait_idx(j1)
                        issue_gadd(b1, j1)

                    wait_gadd(b)
                    issue_scatter(b, k)

                    @pl.when((c >= 2) & (c + IBUF - 2 < NCHUNK))
                    def _():
                        issue_idx(c + IBUF - 2, (k - 2) % IBUF)

            return carry

        lax.fori_loop(0, (NCHUNK + IBUF - 1) // IBUF, group_body, 0)

        for b in range(NBUF):
            wait_scatter(b)

        plsc.subcore_barrier()
        pltpu.sync_copy(acc.at[pl.ds(sid * RPT, RPT)],
                        out_hbm.at[cid, pl.ds(sid * RPT, RPT)])

        @pl.when(sid == NS - 1)
        def _():
            pltpu.sync_copy(acc.at[pl.ds(NS * RPT, REM)],
                            out_hbm.at[cid, pl.ds(NS * RPT, REM)])

    return body(node_feat, edge_feat, src2, dst2, zeros)


BN = 1000


def _mlp_body(agg_ref, w1_ref, b1_ref, w2_ref, b2_ref, out_ref):
    a = agg_ref[0] + agg_ref[1]
    h = jnp.maximum(
        jnp.dot(a, w1_ref[...], preferred_element_type=jnp.float32)
        + b1_ref[...], 0.0)
    out_ref[...] = (
        jnp.dot(h, w2_ref[...], preferred_element_type=jnp.float32)
        + b2_ref[...])


def _mlp(partials, W1, b1, W2, b2):
    return pl.pallas_call(
        _mlp_body,
        grid=(N // BN,),
        in_specs=[
            pl.BlockSpec((NC, BN, D), lambda i: (0, i, 0)),
            pl.BlockSpec((D, H), lambda i: (0, 0)),
            pl.BlockSpec((1, H), lambda i: (0, 0)),
            pl.BlockSpec((H, D), lambda i: (0, 0)),
            pl.BlockSpec((1, D), lambda i: (0, 0)),
        ],
        out_specs=pl.BlockSpec((BN, D), lambda i: (i, 0)),
        out_shape=jax.ShapeDtypeStruct((N, D), jnp.float32),
    )(partials, W1, b1, W2, b2)


@jax.jit
def kernel(node_feat, edge_feat, edge_index, W1, b1, W2, b2):
    src2 = edge_index[0]
    dst2 = edge_index[1]
    zeros = jnp.zeros((N, D), jnp.float32)
    partials = _sc_segment_sum(node_feat, edge_feat, src2, dst2, zeros)
    return _mlp(partials, W1, b1.reshape(1, H), W2, b2.reshape(1, D))
